# Initial kernel scaffold; baseline (speedup 1.0000x reference)
import jax
import jax.numpy as jnp
from jax.experimental import pallas as pl


def kernel(input, emb):
    raise NotImplementedError("write your pallas kernel here")


# speedup vs baseline: 1.7308x; 1.7308x over previous
import jax
import jax.numpy as jnp
from jax.experimental import pallas as pl


_BLOCK_L = 512


def _add_kernel(x_ref, e_ref, o_ref):
    o_ref[...] = x_ref[...] + e_ref[...][None, :, :]


def kernel(input, emb):
    Bv, L, D = input.shape
    grid = (L // _BLOCK_L,)
    return pl.pallas_call(
        _add_kernel,
        grid=grid,
        in_specs=[
            pl.BlockSpec((Bv, _BLOCK_L, D), lambda i: (0, i, 0)),
            pl.BlockSpec((_BLOCK_L, D), lambda i: (i, 0)),
        ],
        out_specs=pl.BlockSpec((Bv, _BLOCK_L, D), lambda i: (0, i, 0)),
        out_shape=jax.ShapeDtypeStruct((Bv, L, D), input.dtype),
    )(input, emb)
